# Initial kernel scaffold; baseline (speedup 1.0000x reference)
import jax
import jax.numpy as jnp
from jax.experimental import pallas as pl


def kernel(x, edge_index, W1, b1, W2, b2):
    raise NotImplementedError("write your pallas kernel here")


# speedup vs baseline: 10.6350x; 10.6350x over previous
import functools

import jax
import jax.numpy as jnp
from jax import lax
from jax.experimental import pallas as pl
from jax.experimental.pallas import tpu as pltpu
from jax.experimental.pallas import tpu_sc as plsc

N = 10000
D = 128
E = 320000

NC = 2
NS = 16
NW = NC * NS

CHUNK = 128
EPT = 10112
NCHUNK = EPT // CHUNK
E_PAD = EPT * NW
N_ACC = 10240

_mesh = plsc.VectorSubcoreMesh(core_axis_name="c", subcore_axis_name="s")


@functools.partial(
    pl.kernel,
    mesh=_mesh,
    out_type=jax.ShapeDtypeStruct((NC, N_ACC), jnp.float32),
    scratch_types=[
        pltpu.VMEM((CHUNK,), jnp.int32),
        pltpu.VMEM((CHUNK,), jnp.float32),
        pltpu.VMEM((N_ACC // NS,), jnp.float32),
        pltpu.VMEM_SHARED((N_ACC,), jnp.float32),
    ],
)
def _deg_kernel(dst_hbm, ones_hbm, zeros_hbm, out_hbm, idx_d, ones_v, obuf,
                acc_sh):
    c = lax.axis_index("c")
    s = lax.axis_index("s")
    wid = c * NS + s
    seg = N_ACC // NS
    pltpu.sync_copy(zeros_hbm, obuf)
    pltpu.sync_copy(obuf, acc_sh.at[pl.ds(s * seg, seg)])
    pltpu.sync_copy(ones_hbm, ones_v)
    plsc.subcore_barrier()

    def body(j, carry):
        base = wid * EPT + j * CHUNK
        pltpu.sync_copy(dst_hbm.at[pl.ds(base, CHUNK)], idx_d)
        pltpu.sync_copy(ones_v, acc_sh.at[idx_d], add=True)
        return carry

    lax.fori_loop(0, NCHUNK, body, 0)
    plsc.subcore_barrier()
    pltpu.sync_copy(acc_sh.at[pl.ds(s * seg, seg)], obuf)
    pltpu.sync_copy(obuf, out_hbm.at[c, pl.ds(s * seg, seg)])


@functools.partial(
    pl.kernel,
    mesh=_mesh,
    out_type=jax.ShapeDtypeStruct((NC, N_ACC, D), jnp.float32),
    scratch_types=[
        pltpu.VMEM((CHUNK,), jnp.int32),
        pltpu.VMEM((CHUNK,), jnp.int32),
        pltpu.VMEM((CHUNK, D), jnp.float32),
        pltpu.VMEM_SHARED((N_ACC, D), jnp.float32),
        pltpu.SemaphoreType.DMA,
    ],
)
def _prop_kernel(y_hbm, src_hbm, dst_hbm, zrow_hbm, out_hbm, idx_s, idx_d,
                 rows, acc_sh, sem):
    c = lax.axis_index("c")
    s = lax.axis_index("s")
    wid = c * NS + s
    pltpu.sync_copy(zrow_hbm, rows)
    for z in range(5):
        pltpu.sync_copy(rows, acc_sh.at[pl.ds((s * 5 + z) * CHUNK, CHUNK)])
    plsc.subcore_barrier()

    def body(j, carry):
        base = wid * EPT + j * CHUNK
        pltpu.sync_copy(src_hbm.at[pl.ds(base, CHUNK)], idx_s)
        pltpu.sync_copy(dst_hbm.at[pl.ds(base, CHUNK)], idx_d)
        pltpu.async_copy(y_hbm.at[idx_s], rows, sem).wait()
        pltpu.sync_copy(rows, acc_sh.at[idx_d], add=True)
        return carry

    lax.fori_loop(0, NCHUNK, body, 0)
    plsc.subcore_barrier()
    for z in range(5):
        r0 = (s * 5 + z) * CHUNK
        pltpu.sync_copy(acc_sh.at[pl.ds(r0, CHUNK)], rows)
        pltpu.sync_copy(rows, out_hbm.at[c, pl.ds(r0, CHUNK)])


def _tc1_body(p0_ref, p1_ref, x_ref, w1_ref, y_ref):
    dis = lax.rsqrt(p0_ref[...] + p1_ref[...] + 1.0)
    h = jnp.dot(x_ref[...], w1_ref[...], preferred_element_type=jnp.float32)
    y_ref[...] = h * dis


def _tc2_body(p0_ref, p1_ref, s0_ref, s1_ref, y1_ref, b1_ref, w2_ref, y2_ref):
    dis = lax.rsqrt(p0_ref[...] + p1_ref[...] + 1.0)
    sagg = s0_ref[pl.ds(0, N), :] + s1_ref[pl.ds(0, N), :]
    z = dis * (sagg + y1_ref[...]) + b1_ref[...]
    h = jnp.maximum(z, 0.0)
    h2 = jnp.dot(h, w2_ref[...], preferred_element_type=jnp.float32)
    y2_ref[...] = h2 * dis


def _tc3_body(p0_ref, p1_ref, s0_ref, s1_ref, y2_ref, b2_ref, out_ref):
    dis = lax.rsqrt(p0_ref[...] + p1_ref[...] + 1.0)
    sagg = s0_ref[pl.ds(0, N), :] + s1_ref[pl.ds(0, N), :]
    out_ref[...] = dis * (sagg + y2_ref[...]) + b2_ref[...]


def kernel(x, edge_index, W1, b1, W2, b2):
    src = edge_index[0]
    dst = edge_index[1]
    pad = E_PAD - E
    src_p = jnp.concatenate([src, jnp.zeros((pad,), jnp.int32)])
    dst_p = jnp.concatenate([dst, jnp.full((pad,), N, jnp.int32)])

    ones_c = jnp.ones((CHUNK,), jnp.float32)
    zeros_seg = jnp.zeros((N_ACC // NS,), jnp.float32)
    zrow = jnp.zeros((CHUNK, D), jnp.float32)

    degp = _deg_kernel(dst_p, ones_c, zeros_seg)
    p0 = degp[0, :N].reshape(N, 1)
    p1 = degp[1, :N].reshape(N, 1)
    b1r = b1.reshape(1, D)
    b2r = b2.reshape(1, D)

    fs = jax.ShapeDtypeStruct((N, D), jnp.float32)
    y1 = pl.pallas_call(_tc1_body, out_shape=fs)(p0, p1, x, W1)
    s1 = _prop_kernel(y1, src_p, dst_p, zrow)
    y2 = pl.pallas_call(_tc2_body, out_shape=fs)(
        p0, p1, s1[0], s1[1], y1, b1r, W2)
    s2 = _prop_kernel(y2, src_p, dst_p, zrow)
    out = pl.pallas_call(_tc3_body, out_shape=fs)(
        p0, p1, s2[0], s2[1], y2, b2r)
    return out
